# baseline (speedup 1.0000x reference)
import functools

import jax
import jax.numpy as jnp
from jax import lax
from jax.experimental import pallas as pl
from jax.experimental.pallas import tpu as pltpu
from jax.experimental.pallas import tpu_sc as plsc

_N, _D, _E = 10000, 256, 160000
_NB = 5
_BR = _N // _NB

_NC, _NS, _L = 2, 16, 16
_NP = 10240
_GARB = 10200
_E2 = 163840
_EPW = _E2 // (_NC * _NS)
_KS = 128
_NKS = _EPW // _KS
_SL = _NP // _NS


def _row_spec(shape):
    return pl.BlockSpec(shape, lambda i: (i, 0))


_FULL_D = pl.BlockSpec((1, _D), lambda i: (0, 0))
_FULL_W = pl.BlockSpec((_D, _D), lambda i: (0, 0))


def _bn_stats_body(x_ref, mean_ref, rstd_ref):
    x = x_ref[...]
    m = jnp.mean(x, axis=0, keepdims=True)
    v = jnp.mean((x - m) * (x - m), axis=0, keepdims=True)
    mean_ref[...] = m
    rstd_ref[...] = lax.rsqrt(v + 1e-5)


def _bn_stats(x):
    return pl.pallas_call(
        _bn_stats_body,
        out_shape=(jax.ShapeDtypeStruct((1, _D), jnp.float32),
                   jax.ShapeDtypeStruct((1, _D), jnp.float32)),
    )(x)


def _le_lin_body(x_ref, mean_ref, rstd_ref, g_ref, bt_ref, w1_ref, b1_ref,
                 w2_ref, w3_ref, b3_ref, a_ref, b_ref, l3_ref):
    xn = (x_ref[...] - mean_ref[...]) * (rstd_ref[...] * g_ref[...]) + bt_ref[...]
    a_ref[...] = jnp.dot(xn, w1_ref[...], preferred_element_type=jnp.float32) + b1_ref[...]
    b_ref[...] = jnp.dot(xn, w2_ref[...], preferred_element_type=jnp.float32)
    l3_ref[...] = jnp.dot(xn, w3_ref[...], preferred_element_type=jnp.float32) + b3_ref[...]


def _le_lin(x, mean, rstd, gamma, beta, w1, b1, w2, w3, b3):
    return pl.pallas_call(
        _le_lin_body,
        grid=(_NB,),
        in_specs=[_row_spec((_BR, _D)), _FULL_D, _FULL_D, _FULL_D, _FULL_D,
                  _FULL_W, _FULL_D, _FULL_W, _FULL_W, _FULL_D],
        out_specs=(_row_spec((_BR, _D)),) * 3,
        out_shape=(jax.ShapeDtypeStruct((_N, _D), jnp.float32),) * 3,
    )(x, mean, rstd, gamma.reshape(1, _D), beta.reshape(1, _D),
      w1, b1.reshape(1, _D), w2, w3, b3.reshape(1, _D))


def _le_comb_body(r1_ref, b_ref, l3_ref, wsum_ref, out_ref):
    out_ref[...] = jnp.maximum(
        r1_ref[...] - b_ref[...] * wsum_ref[...] + l3_ref[...], 0.0)


def _le_comb(r1, b, l3, wsum):
    return pl.pallas_call(
        _le_comb_body,
        grid=(_NB,),
        in_specs=[_row_spec((_BR, _D))] * 3 + [_row_spec((_BR, 1))],
        out_specs=_row_spec((_BR, _D)),
        out_shape=jax.ShapeDtypeStruct((_N, _D), jnp.float32),
    )(r1, b, l3, wsum.reshape(_N, 1))


def _sage_body(relu, r_ref, deg_ref, x_ref, wl_ref, bl_ref, wr_ref, out_ref):
    degc = jnp.maximum(deg_ref[...], 1.0)
    nbr = r_ref[...] / degc
    o = (jnp.dot(nbr, wl_ref[...], preferred_element_type=jnp.float32)
         + bl_ref[...]
         + jnp.dot(x_ref[...], wr_ref[...], preferred_element_type=jnp.float32))
    out_ref[...] = jnp.maximum(o, 0.0) if relu else o


def _sage(r, deg, x, wl, bl, wr, relu):
    return pl.pallas_call(
        functools.partial(_sage_body, relu),
        grid=(_NB,),
        in_specs=[_row_spec((_BR, _D)), _row_spec((_BR, 1)), _row_spec((_BR, _D)),
                  _FULL_W, _FULL_D, _FULL_W],
        out_specs=_row_spec((_BR, _D)),
        out_shape=jax.ShapeDtypeStruct((_N, _D), jnp.float32),
    )(r, deg.reshape(_N, 1), x, wl, bl.reshape(1, _D), wr)


def _gat_pre_body(x_ref, w_ref, asv_ref, adv_ref, h_ref, asrc_ref, adst_ref):
    h = jnp.dot(x_ref[...], w_ref[...], preferred_element_type=jnp.float32)
    h_ref[...] = h
    asrc_ref[...] = jnp.dot(h, asv_ref[...], preferred_element_type=jnp.float32)
    adst_ref[...] = jnp.dot(h, adv_ref[...], preferred_element_type=jnp.float32)


def _gat_pre(x, w, a_src, a_dst):
    return pl.pallas_call(
        _gat_pre_body,
        grid=(_NB,),
        in_specs=[_row_spec((_BR, _D)), _FULL_W,
                  pl.BlockSpec((_D, 1), lambda i: (0, 0)),
                  pl.BlockSpec((_D, 1), lambda i: (0, 0))],
        out_specs=(_row_spec((_BR, _D)), _row_spec((_BR, 1)), _row_spec((_BR, 1))),
        out_shape=(jax.ShapeDtypeStruct((_N, _D), jnp.float32),
                   jax.ShapeDtypeStruct((_N, 1), jnp.float32),
                   jax.ShapeDtypeStruct((_N, 1), jnp.float32)),
    )(x, w, a_src.reshape(_D, 1), a_dst.reshape(_D, 1))


def _lrelu(v):
    return jnp.where(v >= 0, v, 0.2 * v)


def _gat_emax_body(asrc_ref, adst_ref, ep_ref, emax_ref, eeself_ref):
    e_self = _lrelu(asrc_ref[...] + adst_ref[...])
    emax = jnp.maximum(ep_ref[...], e_self)
    emax_ref[...] = emax
    eeself_ref[...] = jnp.exp(e_self - emax)


def _gat_emax_comb(asrc, adst, emax_edge):
    return pl.pallas_call(
        _gat_emax_body,
        out_shape=(jax.ShapeDtypeStruct((_N, 1), jnp.float32),
                   jax.ShapeDtypeStruct((_N, 1), jnp.float32)),
    )(asrc, adst, emax_edge)


def _gat_invd_body(denom_ref, eeself_ref, invd_ref, slf_ref):
    invd = 1.0 / (denom_ref[...] + eeself_ref[...] + 1e-16)
    invd_ref[...] = invd
    slf_ref[...] = eeself_ref[...] * invd


def _gat_invd(denom, eeself):
    return pl.pallas_call(
        _gat_invd_body,
        out_shape=(jax.ShapeDtypeStruct((_N, 1), jnp.float32),
                   jax.ShapeDtypeStruct((_N, 1), jnp.float32)),
    )(denom.reshape(_N, 1), eeself)


def _gat_comb_body(r3_ref, h_ref, invd_ref, slf_ref, gb_ref, out_ref):
    out_ref[...] = jnp.maximum(
        invd_ref[...] * r3_ref[...] + slf_ref[...] * h_ref[...] + gb_ref[...],
        0.0)


def _gat_comb(r3, h, invd, slf, gat_b):
    return pl.pallas_call(
        _gat_comb_body,
        grid=(_NB,),
        in_specs=[_row_spec((_BR, _D)), _row_spec((_BR, _D)),
                  _row_spec((_BR, 1)), _row_spec((_BR, 1)), _FULL_D],
        out_specs=_row_spec((_BR, _D)),
        out_shape=jax.ShapeDtypeStruct((_N, _D), jnp.float32),
    )(r3, h, invd, slf, gat_b.reshape(1, _D))


_MESH = plsc.VectorSubcoreMesh(core_axis_name="c", subcore_axis_name="s",
                               num_cores=_NC, num_subcores=_NS)


def _seg_rows_sc(table, src, dst, scale=None):
    rows = table[src]
    if scale is not None:
        rows = rows * scale[:, None]
    return jax.ops.segment_sum(rows, dst, num_segments=_N)


def _make_stats():
    scratch = [
        pltpu.VMEM_SHARED((_NP,), jnp.float32),
        pltpu.VMEM_SHARED((_NP,), jnp.float32),
        pltpu.VMEM((_NKS, _KS), jnp.int32),
        pltpu.VMEM((_NKS, _KS), jnp.float32),
        pltpu.VMEM((_KS,), jnp.float32),
        pltpu.VMEM((_SL,), jnp.float32),
    ]

    def body(dst2_h, w2_h, deg_h, wsum_h, acc_d, acc_w, dstb, wb, onesb,
             zbuf):
        c = lax.axis_index("c")
        s = lax.axis_index("s")
        wid = c * _NS + s
        zv = jnp.zeros((_L,), jnp.float32)
        ov = jnp.ones((_L,), jnp.float32)

        @pl.loop(0, _SL // _L)
        def _z(v):
            zbuf[pl.ds(v * _L, _L)] = zv

        for k in range(_KS // _L):
            onesb[pl.ds(k * _L, _L)] = ov
        col = pl.ds(s * _SL, _SL)
        pltpu.sync_copy(zbuf, acc_d.at[col])
        pltpu.sync_copy(zbuf, acc_w.at[col])
        plsc.subcore_barrier()

        pltpu.sync_copy(dst2_h.at[pl.ds(wid * _NKS, _NKS)], dstb)
        pltpu.sync_copy(w2_h.at[pl.ds(wid * _NKS, _NKS)], wb)

        @pl.loop(0, _NKS)
        def _chunk(j):
            pltpu.sync_copy(onesb, acc_d.at[dstb.at[j]], add=True)
            pltpu.sync_copy(wb.at[j], acc_w.at[dstb.at[j]], add=True)

        plsc.subcore_barrier()
        pltpu.sync_copy(acc_d.at[col], deg_h.at[c, col])
        pltpu.sync_copy(acc_w.at[col], wsum_h.at[c, col])

    return pl.kernel(
        body,
        out_type=(jax.ShapeDtypeStruct((_NC, _NP), jnp.float32),) * 2,
        mesh=_MESH, scratch_types=scratch,
        compiler_params=pltpu.CompilerParams(needs_layout_passes=False))


def _make_gat_max():
    scratch = [
        pltpu.VMEM_SHARED((_NS, _NP), jnp.float32),
        pltpu.VMEM((_NP,), jnp.float32),
        pltpu.VMEM((_NP,), jnp.float32),
        pltpu.VMEM((_NP + _L,), jnp.float32),
        pltpu.VMEM((_EPW + _L,), jnp.int32),
        pltpu.VMEM((_EPW + _L,), jnp.int32),
        pltpu.VMEM((_EPW + _L,), jnp.float32),
        pltpu.VMEM((_SL,), jnp.float32),
        pltpu.VMEM((_SL,), jnp.float32),
    ]

    def body(src_h, dst_h, asrc_h, adst_h, e_h, emax_h, shard, asrcT, adstT,
             emaxl, srcb, dstb, eb, accb, tmpb):
        c = lax.axis_index("c")
        s = lax.axis_index("s")
        wid = c * _NS + s
        neg = jnp.full((_L,), -1e30, jnp.float32)
        lane0 = jax.lax.iota(jnp.int32, _L) == 0

        @pl.loop(0, _NP // _L + 1)
        def _z(v):
            emaxl[pl.ds(v * _L, _L)] = neg

        pltpu.sync_copy(asrc_h, asrcT)
        pltpu.sync_copy(adst_h, adstT)
        base = wid * _EPW
        pltpu.sync_copy(src_h.at[pl.ds(base, _EPW)], srcb.at[pl.ds(0, _EPW)])
        pltpu.sync_copy(dst_h.at[pl.ds(base, _EPW)], dstb.at[pl.ds(0, _EPW)])

        @pl.loop(0, _EPW // _L)
        def _v(v):
            sl = pl.ds(v * _L, _L)
            a16 = plsc.load_gather(asrcT, [srcb[sl]])
            b16 = plsc.load_gather(adstT, [dstb[sl]])
            vv = a16 + b16
            eb[sl] = jnp.maximum(vv, 0.2 * vv)

        @pl.loop(0, _EPW)
        def _e(e):
            d = dstb[pl.ds(e, _L)][0]
            el = eb[pl.ds(e, _L)][0]
            win = pl.ds(d, _L)
            cur = emaxl[win]
            emaxl[win] = jnp.where(lane0, jnp.maximum(cur, el), cur)

        pltpu.sync_copy(eb.at[pl.ds(0, _EPW)], e_h.at[pl.ds(base, _EPW)])
        pltpu.sync_copy(emaxl.at[pl.ds(0, _NP)], shard.at[s])
        plsc.subcore_barrier()
        col = pl.ds(s * _SL, _SL)
        pltpu.sync_copy(shard.at[0, col], accb)
        for t in range(1, _NS):
            pltpu.sync_copy(shard.at[t, col], tmpb)

            @pl.loop(0, _SL // _L)
            def _acc(v):
                sl = pl.ds(v * _L, _L)
                accb[sl] = jnp.maximum(accb[sl], tmpb[sl])

        pltpu.sync_copy(accb, emax_h.at[c, col])

    return pl.kernel(
        body,
        out_type=(jax.ShapeDtypeStruct((_E2,), jnp.float32),
                  jax.ShapeDtypeStruct((_NC, _NP), jnp.float32)),
        mesh=_MESH, scratch_types=scratch,
        compiler_params=pltpu.CompilerParams(needs_layout_passes=False))


def _make_gat_ee():
    scratch = [
        pltpu.VMEM_SHARED((_NP,), jnp.float32),
        pltpu.VMEM((_NP,), jnp.float32),
        pltpu.VMEM((_NKS, _KS), jnp.int32),
        pltpu.VMEM((_NKS, _KS), jnp.float32),
        pltpu.VMEM((_NKS, _KS), jnp.float32),
        pltpu.VMEM((_SL,), jnp.float32),
    ]

    def body(dst2_h, e2_h, emax_h, ee2_h, den_h, acc, emaxT, dstb, eb, eeb,
             zbuf):
        c = lax.axis_index("c")
        s = lax.axis_index("s")
        wid = c * _NS + s
        zv = jnp.zeros((_L,), jnp.float32)

        @pl.loop(0, _SL // _L)
        def _z(v):
            zbuf[pl.ds(v * _L, _L)] = zv

        col = pl.ds(s * _SL, _SL)
        pltpu.sync_copy(zbuf, acc.at[col])
        plsc.subcore_barrier()

        pltpu.sync_copy(emax_h, emaxT)
        rows = pl.ds(wid * _NKS, _NKS)
        pltpu.sync_copy(dst2_h.at[rows], dstb)
        pltpu.sync_copy(e2_h.at[rows], eb)

        @pl.loop(0, _NKS)
        def _chunk(j):
            for v in range(_KS // _L):
                sl = pl.ds(v * _L, _L)
                m16 = plsc.load_gather(emaxT, [dstb[j, sl]])
                eeb[j, sl] = jnp.exp(eb[j, sl] - m16)
            pltpu.sync_copy(eeb.at[j], acc.at[dstb.at[j]], add=True)

        pltpu.sync_copy(eeb, ee2_h.at[rows])
        plsc.subcore_barrier()
        pltpu.sync_copy(acc.at[col], den_h.at[c, col])

    return pl.kernel(
        body,
        out_type=(jax.ShapeDtypeStruct((_E2 // _KS, _KS), jnp.float32),
                  jax.ShapeDtypeStruct((_NC, _NP), jnp.float32)),
        mesh=_MESH, scratch_types=scratch,
        compiler_params=pltpu.CompilerParams(needs_layout_passes=False))


_sc_stats = _make_stats()
_sc_gat_max = _make_gat_max()
_sc_gat_ee = _make_gat_ee()


def kernel(x, edge_index, edge_weight, bn_gamma, bn_beta, le_w1, le_b1, le_w2,
           le_w3, le_b3, sage2_wl, sage2_bl, sage2_wr, gat_w, gat_a_src,
           gat_a_dst, gat_b, sage4_wl, sage4_bl, sage4_wr):
    src = edge_index[0]
    dst = edge_index[1]
    pad = _E2 - _E
    src_p = jnp.concatenate([src, jnp.zeros((pad,), jnp.int32)])
    dst_p = jnp.concatenate([dst, jnp.full((pad,), _GARB, jnp.int32)])
    w_p = jnp.concatenate([edge_weight, jnp.zeros((pad,), jnp.float32)])
    dst2 = dst_p.reshape(_E2 // _KS, _KS)
    w2 = w_p.reshape(_E2 // _KS, _KS)

    mean, rstd = _bn_stats(x)
    a, b, l3 = _le_lin(x, mean, rstd, bn_gamma, bn_beta, le_w1, le_b1,
                       le_w2, le_w3, le_b3)

    deg_p, wsum_p = _sc_stats(dst2, w2)
    deg = deg_p[0, :_N] + deg_p[1, :_N]
    wsum = wsum_p[0, :_N] + wsum_p[1, :_N]

    r1 = _seg_rows_sc(a, src, dst, scale=edge_weight)
    x1 = _le_comb(r1, b, l3, wsum)

    r2 = _seg_rows_sc(x1, src, dst)
    x2 = _sage(r2, deg, x1, sage2_wl, sage2_bl, sage2_wr, relu=True)

    h, asrc, adst = _gat_pre(x2, gat_w, gat_a_src, gat_a_dst)
    npad = _NP - _N
    asrc_pd = jnp.pad(asrc.reshape(_N), (0, npad))
    adst_pd = jnp.pad(adst.reshape(_N), (0, npad))
    e_edge, emax_p = _sc_gat_max(src_p, dst_p, asrc_pd, adst_pd)
    emax_edge = jnp.maximum(emax_p[0, :_N], emax_p[1, :_N]).reshape(_N, 1)
    emax, eeself = _gat_emax_comb(asrc, adst, emax_edge)
    emax_pd = jnp.pad(emax.reshape(_N), (0, npad))
    ee2, den_p = _sc_gat_ee(dst2, e_edge.reshape(_E2 // _KS, _KS), emax_pd)
    denom = den_p[0, :_N] + den_p[1, :_N]
    invd, slf = _gat_invd(denom, eeself)
    ee = ee2.reshape(_E2)[:_E]
    r3 = _seg_rows_sc(h, src, dst, scale=ee)
    x3 = _gat_comb(r3, h, invd, slf, gat_b)

    r4 = _seg_rows_sc(x3, src, dst)
    out = _sage(r4, deg, x3, sage4_wl, sage4_bl, sage4_wr, relu=False)
    return (out, out)


# speedup vs baseline: 1.0850x; 1.0850x over previous
import functools

import jax
import jax.numpy as jnp
from jax import lax
from jax.experimental import pallas as pl
from jax.experimental.pallas import tpu as pltpu
from jax.experimental.pallas import tpu_sc as plsc

_N, _D, _E = 10000, 256, 160000
_NB = 5
_BR = _N // _NB

_NC, _NS, _L = 2, 16, 16
_NP = 10240
_GARB = 10200
_E2 = 163840
_EPW = _E2 // (_NC * _NS)
_KS = 128
_NKS = _EPW // _KS
_SL = _NP // _NS


def _row_spec(shape):
    return pl.BlockSpec(shape, lambda i: (i, 0))


_FULL_D = pl.BlockSpec((1, _D), lambda i: (0, 0))
_FULL_W = pl.BlockSpec((_D, _D), lambda i: (0, 0))


def _bn_stats_body(x_ref, mean_ref, rstd_ref):
    x = x_ref[...]
    m = jnp.mean(x, axis=0, keepdims=True)
    v = jnp.mean((x - m) * (x - m), axis=0, keepdims=True)
    mean_ref[...] = m
    rstd_ref[...] = lax.rsqrt(v + 1e-5)


def _bn_stats(x):
    return pl.pallas_call(
        _bn_stats_body,
        out_shape=(jax.ShapeDtypeStruct((1, _D), jnp.float32),
                   jax.ShapeDtypeStruct((1, _D), jnp.float32)),
    )(x)


def _le_lin_body(x_ref, mean_ref, rstd_ref, g_ref, bt_ref, w1_ref, b1_ref,
                 w2_ref, w3_ref, b3_ref, a_ref, b_ref, l3_ref):
    xn = (x_ref[...] - mean_ref[...]) * (rstd_ref[...] * g_ref[...]) + bt_ref[...]
    a_ref[...] = jnp.dot(xn, w1_ref[...], preferred_element_type=jnp.float32) + b1_ref[...]
    b_ref[...] = jnp.dot(xn, w2_ref[...], preferred_element_type=jnp.float32)
    l3_ref[...] = jnp.dot(xn, w3_ref[...], preferred_element_type=jnp.float32) + b3_ref[...]


def _le_lin(x, mean, rstd, gamma, beta, w1, b1, w2, w3, b3):
    return pl.pallas_call(
        _le_lin_body,
        grid=(_NB,),
        in_specs=[_row_spec((_BR, _D)), _FULL_D, _FULL_D, _FULL_D, _FULL_D,
                  _FULL_W, _FULL_D, _FULL_W, _FULL_W, _FULL_D],
        out_specs=(_row_spec((_BR, _D)),) * 3,
        out_shape=(jax.ShapeDtypeStruct((_N, _D), jnp.float32),) * 3,
    )(x, mean, rstd, gamma.reshape(1, _D), beta.reshape(1, _D),
      w1, b1.reshape(1, _D), w2, w3, b3.reshape(1, _D))


def _le_comb_body(r1_ref, b_ref, l3_ref, wsum_ref, out_ref):
    out_ref[...] = jnp.maximum(
        r1_ref[...] - b_ref[...] * wsum_ref[...] + l3_ref[...], 0.0)


def _le_comb(r1, b, l3, wsum):
    return pl.pallas_call(
        _le_comb_body,
        grid=(_NB,),
        in_specs=[_row_spec((_BR, _D))] * 3 + [_row_spec((_BR, 1))],
        out_specs=_row_spec((_BR, _D)),
        out_shape=jax.ShapeDtypeStruct((_N, _D), jnp.float32),
    )(r1, b, l3, wsum.reshape(_N, 1))


def _sage_body(relu, r_ref, deg_ref, x_ref, wl_ref, bl_ref, wr_ref, out_ref):
    degc = jnp.maximum(deg_ref[...], 1.0)
    nbr = r_ref[...] / degc
    o = (jnp.dot(nbr, wl_ref[...], preferred_element_type=jnp.float32)
         + bl_ref[...]
         + jnp.dot(x_ref[...], wr_ref[...], preferred_element_type=jnp.float32))
    out_ref[...] = jnp.maximum(o, 0.0) if relu else o


def _sage(r, deg, x, wl, bl, wr, relu):
    return pl.pallas_call(
        functools.partial(_sage_body, relu),
        grid=(_NB,),
        in_specs=[_row_spec((_BR, _D)), _row_spec((_BR, 1)), _row_spec((_BR, _D)),
                  _FULL_W, _FULL_D, _FULL_W],
        out_specs=_row_spec((_BR, _D)),
        out_shape=jax.ShapeDtypeStruct((_N, _D), jnp.float32),
    )(r, deg.reshape(_N, 1), x, wl, bl.reshape(1, _D), wr)


def _gat_pre_body(x_ref, w_ref, asv_ref, adv_ref, h_ref, asrc_ref, adst_ref):
    h = jnp.dot(x_ref[...], w_ref[...], preferred_element_type=jnp.float32)
    h_ref[...] = h
    asrc_ref[...] = jnp.dot(h, asv_ref[...], preferred_element_type=jnp.float32)
    adst_ref[...] = jnp.dot(h, adv_ref[...], preferred_element_type=jnp.float32)


def _gat_pre(x, w, a_src, a_dst):
    return pl.pallas_call(
        _gat_pre_body,
        grid=(_NB,),
        in_specs=[_row_spec((_BR, _D)), _FULL_W,
                  pl.BlockSpec((_D, 1), lambda i: (0, 0)),
                  pl.BlockSpec((_D, 1), lambda i: (0, 0))],
        out_specs=(_row_spec((_BR, _D)), _row_spec((_BR, 1)), _row_spec((_BR, 1))),
        out_shape=(jax.ShapeDtypeStruct((_N, _D), jnp.float32),
                   jax.ShapeDtypeStruct((_N, 1), jnp.float32),
                   jax.ShapeDtypeStruct((_N, 1), jnp.float32)),
    )(x, w, a_src.reshape(_D, 1), a_dst.reshape(_D, 1))


def _lrelu(v):
    return jnp.where(v >= 0, v, 0.2 * v)


def _gat_emax_body(asrc_ref, adst_ref, ep_ref, emax_ref, eeself_ref):
    e_self = _lrelu(asrc_ref[...] + adst_ref[...])
    emax = jnp.maximum(ep_ref[...], e_self)
    emax_ref[...] = emax
    eeself_ref[...] = jnp.exp(e_self - emax)


def _gat_emax_comb(asrc, adst, emax_edge):
    return pl.pallas_call(
        _gat_emax_body,
        out_shape=(jax.ShapeDtypeStruct((_N, 1), jnp.float32),
                   jax.ShapeDtypeStruct((_N, 1), jnp.float32)),
    )(asrc, adst, emax_edge)


def _gat_invd_body(denom_ref, eeself_ref, invd_ref, slf_ref):
    invd = 1.0 / (denom_ref[...] + eeself_ref[...] + 1e-16)
    invd_ref[...] = invd
    slf_ref[...] = eeself_ref[...] * invd


def _gat_invd(denom, eeself):
    return pl.pallas_call(
        _gat_invd_body,
        out_shape=(jax.ShapeDtypeStruct((_N, 1), jnp.float32),
                   jax.ShapeDtypeStruct((_N, 1), jnp.float32)),
    )(denom.reshape(_N, 1), eeself)


def _gat_comb_body(r3_ref, h_ref, invd_ref, slf_ref, gb_ref, out_ref):
    out_ref[...] = jnp.maximum(
        invd_ref[...] * r3_ref[...] + slf_ref[...] * h_ref[...] + gb_ref[...],
        0.0)


def _gat_comb(r3, h, invd, slf, gat_b):
    return pl.pallas_call(
        _gat_comb_body,
        grid=(_NB,),
        in_specs=[_row_spec((_BR, _D)), _row_spec((_BR, _D)),
                  _row_spec((_BR, 1)), _row_spec((_BR, 1)), _FULL_D],
        out_specs=_row_spec((_BR, _D)),
        out_shape=jax.ShapeDtypeStruct((_N, _D), jnp.float32),
    )(r3, h, invd, slf, gat_b.reshape(1, _D))


_MESH = plsc.VectorSubcoreMesh(core_axis_name="c", subcore_axis_name="s",
                               num_cores=_NC, num_subcores=_NS)


_CHS = 40
_EPTS = _E // (_NC * _NS)
_NCHS = _EPTS // _CHS


def _make_row_stage():
    scratch = [
        pltpu.VMEM((_CHS,), jnp.int32),
        pltpu.VMEM((_CHS + _L,), jnp.float32),
        pltpu.VMEM((_CHS, _D), jnp.float32),
        pltpu.SemaphoreType.DMA,
    ]

    def body(tab_h, src_h, sc_h, out_h, srcb, scaleb, rows, sem):
        c = lax.axis_index("c")
        s = lax.axis_index("s")
        wid = c * _NS + s

        @pl.loop(0, _NCHS)
        def _chunk(j):
            base = wid * _EPTS + j * _CHS
            pltpu.sync_copy(src_h.at[pl.ds(base, _CHS)], srcb)
            pltpu.sync_copy(sc_h.at[pl.ds(base, _CHS)],
                            scaleb.at[pl.ds(0, _CHS)])
            pltpu.async_copy(tab_h.at[srcb], rows, sem).wait()

            @pl.loop(0, _CHS)
            def _scale(e):
                sc = scaleb[pl.ds(e, _L)][0]
                for k in range(_D // _L):
                    sl2 = pl.ds(k * _L, _L)
                    rows[e, sl2] = rows[e, sl2] * sc

            pltpu.sync_copy(rows, out_h.at[pl.ds(base, _CHS)])

    return pl.kernel(
        body, out_type=jax.ShapeDtypeStruct((_E, _D), jnp.float32),
        mesh=_MESH, scratch_types=scratch,
        compiler_params=pltpu.CompilerParams(needs_layout_passes=False))


_sc_row_stage = _make_row_stage()


def _seg_rows_sc(table, src, dst, scale=None):
    if scale is not None:
        rows = _sc_row_stage(table, src, scale)
    else:
        rows = table[src]
    return jax.ops.segment_sum(rows, dst, num_segments=_N)


def _make_stats():
    scratch = [
        pltpu.VMEM_SHARED((_NP,), jnp.float32),
        pltpu.VMEM_SHARED((_NP,), jnp.float32),
        pltpu.VMEM((_NKS, _KS), jnp.int32),
        pltpu.VMEM((_NKS, _KS), jnp.float32),
        pltpu.VMEM((_KS,), jnp.float32),
        pltpu.VMEM((_SL,), jnp.float32),
    ]

    def body(dst2_h, w2_h, deg_h, wsum_h, acc_d, acc_w, dstb, wb, onesb,
             zbuf):
        c = lax.axis_index("c")
        s = lax.axis_index("s")
        wid = c * _NS + s
        zv = jnp.zeros((_L,), jnp.float32)
        ov = jnp.ones((_L,), jnp.float32)

        @pl.loop(0, _SL // _L)
        def _z(v):
            zbuf[pl.ds(v * _L, _L)] = zv

        for k in range(_KS // _L):
            onesb[pl.ds(k * _L, _L)] = ov
        col = pl.ds(s * _SL, _SL)
        pltpu.sync_copy(zbuf, acc_d.at[col])
        pltpu.sync_copy(zbuf, acc_w.at[col])
        plsc.subcore_barrier()

        pltpu.sync_copy(dst2_h.at[pl.ds(wid * _NKS, _NKS)], dstb)
        pltpu.sync_copy(w2_h.at[pl.ds(wid * _NKS, _NKS)], wb)

        @pl.loop(0, _NKS)
        def _chunk(j):
            pltpu.sync_copy(onesb, acc_d.at[dstb.at[j]], add=True)
            pltpu.sync_copy(wb.at[j], acc_w.at[dstb.at[j]], add=True)

        plsc.subcore_barrier()
        pltpu.sync_copy(acc_d.at[col], deg_h.at[c, col])
        pltpu.sync_copy(acc_w.at[col], wsum_h.at[c, col])

    return pl.kernel(
        body,
        out_type=(jax.ShapeDtypeStruct((_NC, _NP), jnp.float32),) * 2,
        mesh=_MESH, scratch_types=scratch,
        compiler_params=pltpu.CompilerParams(needs_layout_passes=False))


def _make_gat_max():
    scratch = [
        pltpu.VMEM_SHARED((_NS, _NP), jnp.float32),
        pltpu.VMEM((_NP,), jnp.float32),
        pltpu.VMEM((_NP,), jnp.float32),
        pltpu.VMEM((_NP + _L,), jnp.float32),
        pltpu.VMEM((_EPW + _L,), jnp.int32),
        pltpu.VMEM((_EPW + _L,), jnp.int32),
        pltpu.VMEM((_EPW + _L,), jnp.float32),
        pltpu.VMEM((_SL,), jnp.float32),
        pltpu.VMEM((_SL,), jnp.float32),
    ]

    def body(src_h, dst_h, asrc_h, adst_h, e_h, emax_h, shard, asrcT, adstT,
             emaxl, srcb, dstb, eb, accb, tmpb):
        c = lax.axis_index("c")
        s = lax.axis_index("s")
        wid = c * _NS + s
        neg = jnp.full((_L,), -1e30, jnp.float32)
        lane0 = jax.lax.iota(jnp.int32, _L) == 0

        @pl.loop(0, _NP // _L + 1)
        def _z(v):
            emaxl[pl.ds(v * _L, _L)] = neg

        pltpu.sync_copy(asrc_h, asrcT)
        pltpu.sync_copy(adst_h, adstT)
        base = wid * _EPW
        pltpu.sync_copy(src_h.at[pl.ds(base, _EPW)], srcb.at[pl.ds(0, _EPW)])
        pltpu.sync_copy(dst_h.at[pl.ds(base, _EPW)], dstb.at[pl.ds(0, _EPW)])

        @pl.loop(0, _EPW // _L)
        def _v(v):
            sl = pl.ds(v * _L, _L)
            a16 = plsc.load_gather(asrcT, [srcb[sl]])
            b16 = plsc.load_gather(adstT, [dstb[sl]])
            vv = a16 + b16
            eb[sl] = jnp.maximum(vv, 0.2 * vv)

        @pl.loop(0, _EPW)
        def _e(e):
            d = dstb[pl.ds(e, _L)][0]
            el = eb[pl.ds(e, _L)][0]
            win = pl.ds(d, _L)
            cur = emaxl[win]
            emaxl[win] = jnp.where(lane0, jnp.maximum(cur, el), cur)

        pltpu.sync_copy(eb.at[pl.ds(0, _EPW)], e_h.at[pl.ds(base, _EPW)])
        pltpu.sync_copy(emaxl.at[pl.ds(0, _NP)], shard.at[s])
        plsc.subcore_barrier()
        col = pl.ds(s * _SL, _SL)
        pltpu.sync_copy(shard.at[0, col], accb)
        for t in range(1, _NS):
            pltpu.sync_copy(shard.at[t, col], tmpb)

            @pl.loop(0, _SL // _L)
            def _acc(v):
                sl = pl.ds(v * _L, _L)
                accb[sl] = jnp.maximum(accb[sl], tmpb[sl])

        pltpu.sync_copy(accb, emax_h.at[c, col])

    return pl.kernel(
        body,
        out_type=(jax.ShapeDtypeStruct((_E2,), jnp.float32),
                  jax.ShapeDtypeStruct((_NC, _NP), jnp.float32)),
        mesh=_MESH, scratch_types=scratch,
        compiler_params=pltpu.CompilerParams(needs_layout_passes=False))


def _make_gat_ee():
    scratch = [
        pltpu.VMEM_SHARED((_NP,), jnp.float32),
        pltpu.VMEM((_NP,), jnp.float32),
        pltpu.VMEM((_NKS, _KS), jnp.int32),
        pltpu.VMEM((_NKS, _KS), jnp.float32),
        pltpu.VMEM((_NKS, _KS), jnp.float32),
        pltpu.VMEM((_SL,), jnp.float32),
    ]

    def body(dst2_h, e2_h, emax_h, ee2_h, den_h, acc, emaxT, dstb, eb, eeb,
             zbuf):
        c = lax.axis_index("c")
        s = lax.axis_index("s")
        wid = c * _NS + s
        zv = jnp.zeros((_L,), jnp.float32)

        @pl.loop(0, _SL // _L)
        def _z(v):
            zbuf[pl.ds(v * _L, _L)] = zv

        col = pl.ds(s * _SL, _SL)
        pltpu.sync_copy(zbuf, acc.at[col])
        plsc.subcore_barrier()

        pltpu.sync_copy(emax_h, emaxT)
        rows = pl.ds(wid * _NKS, _NKS)
        pltpu.sync_copy(dst2_h.at[rows], dstb)
        pltpu.sync_copy(e2_h.at[rows], eb)

        @pl.loop(0, _NKS)
        def _chunk(j):
            for v in range(_KS // _L):
                sl = pl.ds(v * _L, _L)
                m16 = plsc.load_gather(emaxT, [dstb[j, sl]])
                eeb[j, sl] = jnp.exp(eb[j, sl] - m16)
            pltpu.sync_copy(eeb.at[j], acc.at[dstb.at[j]], add=True)

        pltpu.sync_copy(eeb, ee2_h.at[rows])
        plsc.subcore_barrier()
        pltpu.sync_copy(acc.at[col], den_h.at[c, col])

    return pl.kernel(
        body,
        out_type=(jax.ShapeDtypeStruct((_E2 // _KS, _KS), jnp.float32),
                  jax.ShapeDtypeStruct((_NC, _NP), jnp.float32)),
        mesh=_MESH, scratch_types=scratch,
        compiler_params=pltpu.CompilerParams(needs_layout_passes=False))


_sc_stats = _make_stats()
_sc_gat_max = _make_gat_max()
_sc_gat_ee = _make_gat_ee()


def kernel(x, edge_index, edge_weight, bn_gamma, bn_beta, le_w1, le_b1, le_w2,
           le_w3, le_b3, sage2_wl, sage2_bl, sage2_wr, gat_w, gat_a_src,
           gat_a_dst, gat_b, sage4_wl, sage4_bl, sage4_wr):
    src = edge_index[0]
    dst = edge_index[1]
    pad = _E2 - _E
    src_p = jnp.concatenate([src, jnp.zeros((pad,), jnp.int32)])
    dst_p = jnp.concatenate([dst, jnp.full((pad,), _GARB, jnp.int32)])
    w_p = jnp.concatenate([edge_weight, jnp.zeros((pad,), jnp.float32)])
    dst2 = dst_p.reshape(_E2 // _KS, _KS)
    w2 = w_p.reshape(_E2 // _KS, _KS)

    mean, rstd = _bn_stats(x)
    a, b, l3 = _le_lin(x, mean, rstd, bn_gamma, bn_beta, le_w1, le_b1,
                       le_w2, le_w3, le_b3)

    deg_p, wsum_p = _sc_stats(dst2, w2)
    deg = deg_p[0, :_N] + deg_p[1, :_N]
    wsum = wsum_p[0, :_N] + wsum_p[1, :_N]

    r1 = _seg_rows_sc(a, src, dst, scale=edge_weight)
    x1 = _le_comb(r1, b, l3, wsum)

    r2 = _seg_rows_sc(x1, src, dst)
    x2 = _sage(r2, deg, x1, sage2_wl, sage2_bl, sage2_wr, relu=True)

    h, asrc, adst = _gat_pre(x2, gat_w, gat_a_src, gat_a_dst)
    npad = _NP - _N
    asrc_pd = jnp.pad(asrc.reshape(_N), (0, npad))
    adst_pd = jnp.pad(adst.reshape(_N), (0, npad))
    e_edge, emax_p = _sc_gat_max(src_p, dst_p, asrc_pd, adst_pd)
    emax_edge = jnp.maximum(emax_p[0, :_N], emax_p[1, :_N]).reshape(_N, 1)
    emax, eeself = _gat_emax_comb(asrc, adst, emax_edge)
    emax_pd = jnp.pad(emax.reshape(_N), (0, npad))
    ee2, den_p = _sc_gat_ee(dst2, e_edge.reshape(_E2 // _KS, _KS), emax_pd)
    denom = den_p[0, :_N] + den_p[1, :_N]
    invd, slf = _gat_invd(denom, eeself)
    ee = ee2.reshape(_E2)[:_E]
    r3 = _seg_rows_sc(h, src, dst, scale=ee)
    x3 = _gat_comb(r3, h, invd, slf, gat_b)

    r4 = _seg_rows_sc(x3, src, dst)
    out = _sage(r4, deg, x3, sage4_wl, sage4_bl, sage4_wr, relu=False)
    return (out, out)


# speedup vs baseline: 1.1418x; 1.0523x over previous
import functools

import jax
import jax.numpy as jnp
from jax import lax
from jax.experimental import pallas as pl
from jax.experimental.pallas import tpu as pltpu
from jax.experimental.pallas import tpu_sc as plsc

_N, _D, _E = 10000, 256, 160000
_NB = 5
_BR = _N // _NB

_NC, _NS, _L = 2, 16, 16
_NP = 10240
_GARB = 10200
_E2 = 163840
_EPW = _E2 // (_NC * _NS)
_KS = 128
_NKS = _EPW // _KS
_SL = _NP // _NS


def _row_spec(shape):
    return pl.BlockSpec(shape, lambda i: (i, 0))


_FULL_D = pl.BlockSpec((1, _D), lambda i: (0, 0))
_FULL_W = pl.BlockSpec((_D, _D), lambda i: (0, 0))


def _bn_stats_body(x_ref, mean_ref, rstd_ref):
    x = x_ref[...]
    m = jnp.mean(x, axis=0, keepdims=True)
    v = jnp.mean((x - m) * (x - m), axis=0, keepdims=True)
    mean_ref[...] = m
    rstd_ref[...] = lax.rsqrt(v + 1e-5)


def _bn_stats(x):
    return pl.pallas_call(
        _bn_stats_body,
        out_shape=(jax.ShapeDtypeStruct((1, _D), jnp.float32),
                   jax.ShapeDtypeStruct((1, _D), jnp.float32)),
    )(x)


def _le_lin_body(x_ref, mean_ref, rstd_ref, g_ref, bt_ref, w1_ref, b1_ref,
                 w2_ref, w3_ref, b3_ref, a_ref, b_ref, l3_ref):
    xn = (x_ref[...] - mean_ref[...]) * (rstd_ref[...] * g_ref[...]) + bt_ref[...]
    a_ref[...] = jnp.dot(xn, w1_ref[...], preferred_element_type=jnp.float32) + b1_ref[...]
    b_ref[...] = jnp.dot(xn, w2_ref[...], preferred_element_type=jnp.float32)
    l3_ref[...] = jnp.dot(xn, w3_ref[...], preferred_element_type=jnp.float32) + b3_ref[...]


def _le_lin(x, mean, rstd, gamma, beta, w1, b1, w2, w3, b3):
    return pl.pallas_call(
        _le_lin_body,
        grid=(_NB,),
        in_specs=[_row_spec((_BR, _D)), _FULL_D, _FULL_D, _FULL_D, _FULL_D,
                  _FULL_W, _FULL_D, _FULL_W, _FULL_W, _FULL_D],
        out_specs=(_row_spec((_BR, _D)),) * 3,
        out_shape=(jax.ShapeDtypeStruct((_N, _D), jnp.float32),) * 3,
    )(x, mean, rstd, gamma.reshape(1, _D), beta.reshape(1, _D),
      w1, b1.reshape(1, _D), w2, w3, b3.reshape(1, _D))


def _le_comb_body(r1_ref, b_ref, l3_ref, wsum_ref, out_ref):
    out_ref[...] = jnp.maximum(
        r1_ref[...] - b_ref[...] * wsum_ref[...] + l3_ref[...], 0.0)


def _le_comb(r1, b, l3, wsum):
    return pl.pallas_call(
        _le_comb_body,
        grid=(_NB,),
        in_specs=[_row_spec((_BR, _D))] * 3 + [_row_spec((_BR, 1))],
        out_specs=_row_spec((_BR, _D)),
        out_shape=jax.ShapeDtypeStruct((_N, _D), jnp.float32),
    )(r1, b, l3, wsum.reshape(_N, 1))


def _sage_body(relu, r_ref, deg_ref, x_ref, wl_ref, bl_ref, wr_ref, out_ref):
    degc = jnp.maximum(deg_ref[...], 1.0)
    nbr = r_ref[...] / degc
    o = (jnp.dot(nbr, wl_ref[...], preferred_element_type=jnp.float32)
         + bl_ref[...]
         + jnp.dot(x_ref[...], wr_ref[...], preferred_element_type=jnp.float32))
    out_ref[...] = jnp.maximum(o, 0.0) if relu else o


def _sage(r, deg, x, wl, bl, wr, relu):
    return pl.pallas_call(
        functools.partial(_sage_body, relu),
        grid=(_NB,),
        in_specs=[_row_spec((_BR, _D)), _row_spec((_BR, 1)), _row_spec((_BR, _D)),
                  _FULL_W, _FULL_D, _FULL_W],
        out_specs=_row_spec((_BR, _D)),
        out_shape=jax.ShapeDtypeStruct((_N, _D), jnp.float32),
    )(r, deg.reshape(_N, 1), x, wl, bl.reshape(1, _D), wr)


def _gat_pre_body(x_ref, w_ref, asv_ref, adv_ref, h_ref, asrc_ref, adst_ref):
    h = jnp.dot(x_ref[...], w_ref[...], preferred_element_type=jnp.float32)
    h_ref[...] = h
    asrc_ref[...] = jnp.dot(h, asv_ref[...], preferred_element_type=jnp.float32)
    adst_ref[...] = jnp.dot(h, adv_ref[...], preferred_element_type=jnp.float32)


def _gat_pre(x, w, a_src, a_dst):
    return pl.pallas_call(
        _gat_pre_body,
        grid=(_NB,),
        in_specs=[_row_spec((_BR, _D)), _FULL_W,
                  pl.BlockSpec((_D, 1), lambda i: (0, 0)),
                  pl.BlockSpec((_D, 1), lambda i: (0, 0))],
        out_specs=(_row_spec((_BR, _D)), _row_spec((_BR, 1)), _row_spec((_BR, 1))),
        out_shape=(jax.ShapeDtypeStruct((_N, _D), jnp.float32),
                   jax.ShapeDtypeStruct((_N, 1), jnp.float32),
                   jax.ShapeDtypeStruct((_N, 1), jnp.float32)),
    )(x, w, a_src.reshape(_D, 1), a_dst.reshape(_D, 1))


def _lrelu(v):
    return jnp.where(v >= 0, v, 0.2 * v)


def _gat_emax_body(asrc_ref, adst_ref, ep_ref, emax_ref, eeself_ref):
    e_self = _lrelu(asrc_ref[...] + adst_ref[...])
    emax = jnp.maximum(ep_ref[...], e_self)
    emax_ref[...] = emax
    eeself_ref[...] = jnp.exp(e_self - emax)


def _gat_emax_comb(asrc, adst, emax_edge):
    return pl.pallas_call(
        _gat_emax_body,
        out_shape=(jax.ShapeDtypeStruct((_N, 1), jnp.float32),
                   jax.ShapeDtypeStruct((_N, 1), jnp.float32)),
    )(asrc, adst, emax_edge)


def _gat_invd_body(denom_ref, eeself_ref, invd_ref, slf_ref):
    invd = 1.0 / (denom_ref[...] + eeself_ref[...] + 1e-16)
    invd_ref[...] = invd
    slf_ref[...] = eeself_ref[...] * invd


def _gat_invd(denom, eeself):
    return pl.pallas_call(
        _gat_invd_body,
        out_shape=(jax.ShapeDtypeStruct((_N, 1), jnp.float32),
                   jax.ShapeDtypeStruct((_N, 1), jnp.float32)),
    )(denom.reshape(_N, 1), eeself)


def _gat_comb_body(r3_ref, h_ref, invd_ref, slf_ref, gb_ref, out_ref):
    out_ref[...] = jnp.maximum(
        invd_ref[...] * r3_ref[...] + slf_ref[...] * h_ref[...] + gb_ref[...],
        0.0)


def _gat_comb(r3, h, invd, slf, gat_b):
    return pl.pallas_call(
        _gat_comb_body,
        grid=(_NB,),
        in_specs=[_row_spec((_BR, _D)), _row_spec((_BR, _D)),
                  _row_spec((_BR, 1)), _row_spec((_BR, 1)), _FULL_D],
        out_specs=_row_spec((_BR, _D)),
        out_shape=jax.ShapeDtypeStruct((_N, _D), jnp.float32),
    )(r3, h, invd, slf, gat_b.reshape(1, _D))


_MESH = plsc.VectorSubcoreMesh(core_axis_name="c", subcore_axis_name="s",
                               num_cores=_NC, num_subcores=_NS)


_CHS = 40
_EPTS = _E // (_NC * _NS)
_NCHS = _EPTS // _CHS


def _make_row_stage(scaled):
    scratch = [
        pltpu.VMEM((_CHS,), jnp.int32),
        pltpu.VMEM((_CHS + _L,), jnp.float32),
        pltpu.VMEM((_CHS, _D), jnp.float32),
        pltpu.SemaphoreType.DMA,
    ]

    def body(*refs):
        if scaled:
            tab_h, src_h, sc_h, out_h, srcb, scaleb, rows, sem = refs
        else:
            tab_h, src_h, out_h, srcb, scaleb, rows, sem = refs
        c = lax.axis_index("c")
        s = lax.axis_index("s")
        wid = c * _NS + s

        @pl.loop(0, _NCHS)
        def _chunk(j):
            base = wid * _EPTS + j * _CHS
            pltpu.sync_copy(src_h.at[pl.ds(base, _CHS)], srcb)
            if scaled:
                pltpu.sync_copy(sc_h.at[pl.ds(base, _CHS)],
                                scaleb.at[pl.ds(0, _CHS)])
            pltpu.async_copy(tab_h.at[srcb], rows, sem).wait()
            if scaled:
                @pl.loop(0, _CHS)
                def _scale(e):
                    sc = scaleb[pl.ds(e, _L)][0]
                    for k in range(_D // _L):
                        sl2 = pl.ds(k * _L, _L)
                        rows[e, sl2] = rows[e, sl2] * sc

            pltpu.sync_copy(rows, out_h.at[pl.ds(base, _CHS)])

    return pl.kernel(
        body, out_type=jax.ShapeDtypeStruct((_E, _D), jnp.float32),
        mesh=_MESH, scratch_types=scratch,
        compiler_params=pltpu.CompilerParams(needs_layout_passes=False))


_sc_row_stage_scaled = _make_row_stage(True)
_sc_row_stage_plain = _make_row_stage(False)


def _seg_rows_sc(table, src, dst, scale=None):
    if scale is not None:
        rows = _sc_row_stage_scaled(table, src, scale)
    else:
        rows = _sc_row_stage_plain(table, src)
    return jax.ops.segment_sum(rows, dst, num_segments=_N)


def _make_stats():
    scratch = [
        pltpu.VMEM_SHARED((_NP,), jnp.float32),
        pltpu.VMEM_SHARED((_NP,), jnp.float32),
        pltpu.VMEM((_NKS, _KS), jnp.int32),
        pltpu.VMEM((_NKS, _KS), jnp.float32),
        pltpu.VMEM((_KS,), jnp.float32),
        pltpu.VMEM((_SL,), jnp.float32),
    ]

    def body(dst2_h, w2_h, deg_h, wsum_h, acc_d, acc_w, dstb, wb, onesb,
             zbuf):
        c = lax.axis_index("c")
        s = lax.axis_index("s")
        wid = c * _NS + s
        zv = jnp.zeros((_L,), jnp.float32)
        ov = jnp.ones((_L,), jnp.float32)

        @pl.loop(0, _SL // _L)
        def _z(v):
            zbuf[pl.ds(v * _L, _L)] = zv

        for k in range(_KS // _L):
            onesb[pl.ds(k * _L, _L)] = ov
        col = pl.ds(s * _SL, _SL)
        pltpu.sync_copy(zbuf, acc_d.at[col])
        pltpu.sync_copy(zbuf, acc_w.at[col])
        plsc.subcore_barrier()

        pltpu.sync_copy(dst2_h.at[pl.ds(wid * _NKS, _NKS)], dstb)
        pltpu.sync_copy(w2_h.at[pl.ds(wid * _NKS, _NKS)], wb)

        @pl.loop(0, _NKS)
        def _chunk(j):
            pltpu.sync_copy(onesb, acc_d.at[dstb.at[j]], add=True)
            pltpu.sync_copy(wb.at[j], acc_w.at[dstb.at[j]], add=True)

        plsc.subcore_barrier()
        pltpu.sync_copy(acc_d.at[col], deg_h.at[c, col])
        pltpu.sync_copy(acc_w.at[col], wsum_h.at[c, col])

    return pl.kernel(
        body,
        out_type=(jax.ShapeDtypeStruct((_NC, _NP), jnp.float32),) * 2,
        mesh=_MESH, scratch_types=scratch,
        compiler_params=pltpu.CompilerParams(needs_layout_passes=False))


def _make_gat_max():
    scratch = [
        pltpu.VMEM_SHARED((_NS, _NP), jnp.float32),
        pltpu.VMEM((_NP,), jnp.float32),
        pltpu.VMEM((_NP,), jnp.float32),
        pltpu.VMEM((_NP + _L,), jnp.float32),
        pltpu.VMEM((_EPW + _L,), jnp.int32),
        pltpu.VMEM((_EPW + _L,), jnp.int32),
        pltpu.VMEM((_EPW + _L,), jnp.float32),
        pltpu.VMEM((_SL,), jnp.float32),
        pltpu.VMEM((_SL,), jnp.float32),
    ]

    def body(src_h, dst_h, asrc_h, adst_h, e_h, emax_h, shard, asrcT, adstT,
             emaxl, srcb, dstb, eb, accb, tmpb):
        c = lax.axis_index("c")
        s = lax.axis_index("s")
        wid = c * _NS + s
        neg = jnp.full((_L,), -1e30, jnp.float32)
        lane0 = jax.lax.iota(jnp.int32, _L) == 0

        @pl.loop(0, _NP // _L + 1)
        def _z(v):
            emaxl[pl.ds(v * _L, _L)] = neg

        pltpu.sync_copy(asrc_h, asrcT)
        pltpu.sync_copy(adst_h, adstT)
        base = wid * _EPW
        pltpu.sync_copy(src_h.at[pl.ds(base, _EPW)], srcb.at[pl.ds(0, _EPW)])
        pltpu.sync_copy(dst_h.at[pl.ds(base, _EPW)], dstb.at[pl.ds(0, _EPW)])

        @pl.loop(0, _EPW // _L)
        def _v(v):
            sl = pl.ds(v * _L, _L)
            a16 = plsc.load_gather(asrcT, [srcb[sl]])
            b16 = plsc.load_gather(adstT, [dstb[sl]])
            vv = a16 + b16
            eb[sl] = jnp.maximum(vv, 0.2 * vv)

        @pl.loop(0, _EPW)
        def _e(e):
            d = dstb[pl.ds(e, _L)][0]
            el = eb[pl.ds(e, _L)][0]
            win = pl.ds(d, _L)
            cur = emaxl[win]
            emaxl[win] = jnp.where(lane0, jnp.maximum(cur, el), cur)

        pltpu.sync_copy(eb.at[pl.ds(0, _EPW)], e_h.at[pl.ds(base, _EPW)])
        pltpu.sync_copy(emaxl.at[pl.ds(0, _NP)], shard.at[s])
        plsc.subcore_barrier()
        col = pl.ds(s * _SL, _SL)
        pltpu.sync_copy(shard.at[0, col], accb)
        for t in range(1, _NS):
            pltpu.sync_copy(shard.at[t, col], tmpb)

            @pl.loop(0, _SL // _L)
            def _acc(v):
                sl = pl.ds(v * _L, _L)
                accb[sl] = jnp.maximum(accb[sl], tmpb[sl])

        pltpu.sync_copy(accb, emax_h.at[c, col])

    return pl.kernel(
        body,
        out_type=(jax.ShapeDtypeStruct((_E2,), jnp.float32),
                  jax.ShapeDtypeStruct((_NC, _NP), jnp.float32)),
        mesh=_MESH, scratch_types=scratch,
        compiler_params=pltpu.CompilerParams(needs_layout_passes=False))


def _make_gat_ee():
    scratch = [
        pltpu.VMEM_SHARED((_NP,), jnp.float32),
        pltpu.VMEM((_NP,), jnp.float32),
        pltpu.VMEM((_NKS, _KS), jnp.int32),
        pltpu.VMEM((_NKS, _KS), jnp.float32),
        pltpu.VMEM((_NKS, _KS), jnp.float32),
        pltpu.VMEM((_SL,), jnp.float32),
    ]

    def body(dst2_h, e2_h, emax_h, ee2_h, den_h, acc, emaxT, dstb, eb, eeb,
             zbuf):
        c = lax.axis_index("c")
        s = lax.axis_index("s")
        wid = c * _NS + s
        zv = jnp.zeros((_L,), jnp.float32)

        @pl.loop(0, _SL // _L)
        def _z(v):
            zbuf[pl.ds(v * _L, _L)] = zv

        col = pl.ds(s * _SL, _SL)
        pltpu.sync_copy(zbuf, acc.at[col])
        plsc.subcore_barrier()

        pltpu.sync_copy(emax_h, emaxT)
        rows = pl.ds(wid * _NKS, _NKS)
        pltpu.sync_copy(dst2_h.at[rows], dstb)
        pltpu.sync_copy(e2_h.at[rows], eb)

        @pl.loop(0, _NKS)
        def _chunk(j):
            for v in range(_KS // _L):
                sl = pl.ds(v * _L, _L)
                m16 = plsc.load_gather(emaxT, [dstb[j, sl]])
                eeb[j, sl] = jnp.exp(eb[j, sl] - m16)
            pltpu.sync_copy(eeb.at[j], acc.at[dstb.at[j]], add=True)

        pltpu.sync_copy(eeb, ee2_h.at[rows])
        plsc.subcore_barrier()
        pltpu.sync_copy(acc.at[col], den_h.at[c, col])

    return pl.kernel(
        body,
        out_type=(jax.ShapeDtypeStruct((_E2 // _KS, _KS), jnp.float32),
                  jax.ShapeDtypeStruct((_NC, _NP), jnp.float32)),
        mesh=_MESH, scratch_types=scratch,
        compiler_params=pltpu.CompilerParams(needs_layout_passes=False))


_sc_stats = _make_stats()
_sc_gat_max = _make_gat_max()
_sc_gat_ee = _make_gat_ee()


def kernel(x, edge_index, edge_weight, bn_gamma, bn_beta, le_w1, le_b1, le_w2,
           le_w3, le_b3, sage2_wl, sage2_bl, sage2_wr, gat_w, gat_a_src,
           gat_a_dst, gat_b, sage4_wl, sage4_bl, sage4_wr):
    src = edge_index[0]
    dst = edge_index[1]
    pad = _E2 - _E
    src_p = jnp.concatenate([src, jnp.zeros((pad,), jnp.int32)])
    dst_p = jnp.concatenate([dst, jnp.full((pad,), _GARB, jnp.int32)])
    w_p = jnp.concatenate([edge_weight, jnp.zeros((pad,), jnp.float32)])
    dst2 = dst_p.reshape(_E2 // _KS, _KS)
    w2 = w_p.reshape(_E2 // _KS, _KS)

    mean, rstd = _bn_stats(x)
    a, b, l3 = _le_lin(x, mean, rstd, bn_gamma, bn_beta, le_w1, le_b1,
                       le_w2, le_w3, le_b3)

    deg_p, wsum_p = _sc_stats(dst2, w2)
    deg = deg_p[0, :_N] + deg_p[1, :_N]
    wsum = wsum_p[0, :_N] + wsum_p[1, :_N]

    r1 = _seg_rows_sc(a, src, dst, scale=edge_weight)
    x1 = _le_comb(r1, b, l3, wsum)

    r2 = _seg_rows_sc(x1, src, dst)
    x2 = _sage(r2, deg, x1, sage2_wl, sage2_bl, sage2_wr, relu=True)

    h, asrc, adst = _gat_pre(x2, gat_w, gat_a_src, gat_a_dst)
    npad = _NP - _N
    asrc_pd = jnp.pad(asrc.reshape(_N), (0, npad))
    adst_pd = jnp.pad(adst.reshape(_N), (0, npad))
    e_edge, emax_p = _sc_gat_max(src_p, dst_p, asrc_pd, adst_pd)
    emax_edge = jnp.maximum(emax_p[0, :_N], emax_p[1, :_N]).reshape(_N, 1)
    emax, eeself = _gat_emax_comb(asrc, adst, emax_edge)
    emax_pd = jnp.pad(emax.reshape(_N), (0, npad))
    ee2, den_p = _sc_gat_ee(dst2, e_edge.reshape(_E2 // _KS, _KS), emax_pd)
    denom = den_p[0, :_N] + den_p[1, :_N]
    invd, slf = _gat_invd(denom, eeself)
    ee = ee2.reshape(_E2)[:_E]
    r3 = _seg_rows_sc(h, src, dst, scale=ee)
    x3 = _gat_comb(r3, h, invd, slf, gat_b)

    r4 = _seg_rows_sc(x3, src, dst)
    out = _sage(r4, deg, x3, sage4_wl, sage4_bl, sage4_wr, relu=False)
    return (out, out)
